# baseline (speedup 1.0000x reference)
import jax
import jax.numpy as jnp
from jax import lax
from jax.experimental import pallas as pl
from jax.experimental.pallas import tpu as pltpu
from jax.experimental.pallas import tpu_sc as plsc

N_NODES = 10000
N_EDGES = 320000
D_FEAT = 128
HALF = D_FEAT // 2

NC = 2
NS = 16
CH = 128
CPT = 160
E_PAD = NS * CPT * CH
NB = 8
LEAD = 6
SRC_CPT = CPT + LEAD
N_RPAD = 10240
ROWS_PT = N_RPAD // NS
FB = 64
NFB = ROWS_PT // FB
N_PAD = N_RPAD


def _sc_body(xs, idxb, eps16, out, acc, xb, ab, epsv, *ring):
  bufs = ring[:NB]
  idxs = ring[NB:2 * NB]
  isem = ring[2 * NB:3 * NB]
  gsem = ring[3 * NB:4 * NB]
  ssem = ring[4 * NB:5 * NB]
  c = lax.axis_index("c")
  s = lax.axis_index("s")
  row0 = s * ROWS_PT

  def stage_idx(j, b, guard):
    if guard:
      pltpu.make_async_copy(bufs[b], acc.at[idxs[b].at[1]], ssem[b]).wait()
    pltpu.make_async_copy(idxb.at[s, j], idxs[b], isem[b]).start()

  def stage_gather(j, b):
    pltpu.make_async_copy(idxb.at[s, j], idxs[b], isem[b]).wait()
    pltpu.make_async_copy(xs.at[c].at[idxs[b].at[0]], bufs[b], gsem[b]).start()

  def stage_scatter(j, b):
    pltpu.make_async_copy(xs.at[c].at[idxs[b].at[0]], bufs[b], gsem[b]).wait()
    pltpu.async_copy(bufs[b], acc.at[idxs[b].at[1]], ssem[b], add=True)

  pltpu.sync_copy(eps16, epsv)
  for b in range(NFB):
    r0 = row0 + b * FB
    pltpu.sync_copy(xs.at[c, pl.ds(r0, FB)], xb)
    pltpu.sync_copy(xb, acc.at[pl.ds(r0, FB)])
  plsc.subcore_barrier()

  for q in range(LEAD):
    stage_idx(q, q, False)
  for q in range(LEAD - 1):
    stage_gather(q, q)

  for b in range(NB):
    stage_idx(b + LEAD, (b + LEAD) % NB, b >= NB - LEAD)
    stage_gather(b + LEAD - 1, (b + LEAD - 1) % NB)
    stage_scatter(b, b)

  def edge_body(i, carry):
    for b in range(NB):
      j = NB * i + b
      stage_idx(j + LEAD, (b + LEAD) % NB, True)
      stage_gather(j + LEAD - 1, (b + LEAD - 1) % NB)
      stage_scatter(j, b)
    return carry

  lax.fori_loop(1, CPT // NB, edge_body, 0)

  for j in range(CPT - (NB - LEAD), CPT):
    b = j % NB
    pltpu.make_async_copy(bufs[b], acc.at[idxs[b].at[1]], ssem[b]).wait()
  for j in range(CPT, CPT + LEAD - 1):
    b = j % NB
    pltpu.make_async_copy(xs.at[c].at[idxs[b].at[0]], bufs[b], gsem[b]).wait()
  b = (CPT + LEAD - 1) % NB
  pltpu.make_async_copy(idxb.at[s, CPT + LEAD - 1], idxs[b], isem[b]).wait()
  plsc.subcore_barrier()

  ev = epsv[...]
  for b in range(NFB):
    r0 = row0 + b * FB
    pltpu.sync_copy(acc.at[pl.ds(r0, FB)], ab)
    pltpu.sync_copy(xs.at[c, pl.ds(r0, FB)], xb)

    def row_body(i, carry):
      arow = ab.at[i]
      xrow = xb.at[i]
      for k in range(HALF // 16):
        sl = pl.ds(k * 16, 16)
        arow[sl] = arow[sl] + ev * xrow[sl]
      return carry

    lax.fori_loop(0, FB, row_body, 0)
    pltpu.sync_copy(ab, out.at[c, pl.ds(r0, FB)])


@jax.jit
def kernel(graph, x, eps):
  graph = graph.astype(jnp.int32)
  src = graph[0]
  dst = graph[1]
  pad_s = jnp.zeros((E_PAD - N_EDGES,), jnp.int32)
  srcp = jnp.concatenate([src, pad_s]).reshape(NS, CPT, CH)
  srcp = jnp.concatenate([srcp, jnp.zeros((NS, LEAD, CH), jnp.int32)], axis=1)
  pad_d = jnp.full((E_PAD - N_EDGES,), N_NODES, jnp.int32)
  dstp = jnp.concatenate([dst, pad_d]).reshape(NS, CPT, CH)
  dstp = jnp.concatenate(
      [dstp, jnp.full((NS, LEAD, CH), N_NODES, jnp.int32)], axis=1)
  idxb = jnp.stack([srcp, dstp], axis=2)
  xp = jnp.concatenate([x, jnp.zeros((N_RPAD - N_NODES, D_FEAT), x.dtype)])
  xs = jnp.stack([xp[:, :HALF], xp[:, HALF:]])
  eps16 = jnp.broadcast_to(eps.astype(jnp.float32), (16,))

  fn = pl.kernel(
      _sc_body,
      out_type=jax.ShapeDtypeStruct((NC, N_RPAD, HALF), jnp.float32),
      mesh=plsc.VectorSubcoreMesh(core_axis_name="c", subcore_axis_name="s"),
      compiler_params=pltpu.CompilerParams(use_tc_tiling_on_sc=False),
      scratch_types=[
          pltpu.VMEM_SHARED((N_PAD, HALF), jnp.float32),
          pltpu.VMEM((FB, HALF), jnp.float32),
          pltpu.VMEM((FB, HALF), jnp.float32),
          pltpu.VMEM((16,), jnp.float32),
      ] + [pltpu.VMEM((CH, HALF), jnp.float32)] * NB
        + [pltpu.VMEM((2, CH), jnp.int32)] * NB
        + [pltpu.SemaphoreType.DMA] * (3 * NB),
  )
  o = fn(xs, idxb, eps16)
  return o.transpose(1, 0, 2).reshape(N_RPAD, D_FEAT)[:N_NODES]
